# Initial kernel scaffold; baseline (speedup 1.0000x reference)
import jax
import jax.numpy as jnp
from jax import lax
from jax.experimental import pallas as pl
from jax.experimental.pallas import tpu as pltpu
from jax.experimental.pallas import tpu_sc as plsc

N = 10000
E = 320000
IN_CH = 128
HID = 32
NC, NS = 2, 16
NW = NC * NS
N_PAD = 10240
RPT = N_PAD // NS
EPT = E // NW
CHUNK = 128
FULL_CHUNKS = EPT // CHUNK
TAIL = EPT - FULL_CHUNKS * CHUNK
BR = 2000

_mesh = plsc.VectorSubcoreMesh(core_axis_name="c", subcore_axis_name="s")


def _deg_body(ei, degp, ones_v, ones_t, didx, didx_t, zrow, deg_sh):
    c = lax.axis_index("c")
    s = lax.axis_index("s")
    wid = c * NS + s
    one16 = jnp.full((16,), 1.0, jnp.float32)
    zero16 = jnp.zeros((16,), jnp.float32)
    for j in range(CHUNK // 16):
        ones_v[pl.ds(j * 16, 16)] = one16
    ones_t[pl.ds(0, 16)] = one16

    @pl.loop(0, RPT // 16)
    def _(i):
        zrow[pl.ds(i * 16, 16)] = zero16

    pltpu.sync_copy(zrow, deg_sh.at[pl.ds(s * RPT, RPT)])
    plsc.subcore_barrier()

    base0 = wid * EPT

    @pl.loop(0, FULL_CHUNKS)
    def _(ch):
        b = base0 + ch * CHUNK
        pltpu.sync_copy(ei.at[1, pl.ds(b, CHUNK)], didx)
        pltpu.sync_copy(ones_v, deg_sh.at[didx], add=True)

    bt = base0 + FULL_CHUNKS * CHUNK
    pltpu.sync_copy(ei.at[1, pl.ds(bt, TAIL)], didx_t)
    pltpu.sync_copy(ones_t, deg_sh.at[didx_t], add=True)

    plsc.subcore_barrier()
    pltpu.sync_copy(deg_sh.at[pl.ds(s * RPT, RPT)], zrow)
    pltpu.sync_copy(zrow, degp.at[c, pl.ds(s * RPT, RPT)])


_deg_call = pl.kernel(
    _deg_body,
    out_type=jax.ShapeDtypeStruct((NC, N_PAD), jnp.float32),
    mesh=_mesh,
    scratch_types=[
        pltpu.VMEM((CHUNK,), jnp.float32),
        pltpu.VMEM((TAIL,), jnp.float32),
        pltpu.VMEM((CHUNK,), jnp.int32),
        pltpu.VMEM((TAIL,), jnp.int32),
        pltpu.VMEM((RPT,), jnp.float32),
        pltpu.VMEM_SHARED((N_PAD,), jnp.float32),
    ],
)


def _prop_body(h, ei, accp, sidx, sidx_t, didx, didx_t, rows, rows_t, fl,
               acc_sh, sem):
    c = lax.axis_index("c")
    s = lax.axis_index("s")
    wid = c * NS + s
    zero16 = jnp.zeros((16,), jnp.float32)

    @pl.loop(0, RPT)
    def _(i):
        fl[i, pl.ds(0, 16)] = zero16
        fl[i, pl.ds(16, 16)] = zero16

    pltpu.sync_copy(fl, acc_sh.at[pl.ds(s * RPT, RPT)])
    plsc.subcore_barrier()

    base0 = wid * EPT

    @pl.loop(0, FULL_CHUNKS)
    def _(ch):
        b = base0 + ch * CHUNK
        pltpu.sync_copy(ei.at[0, pl.ds(b, CHUNK)], sidx)
        pltpu.sync_copy(ei.at[1, pl.ds(b, CHUNK)], didx)
        pltpu.async_copy(h.at[sidx], rows, sem).wait()
        pltpu.sync_copy(rows, acc_sh.at[didx], add=True)

    bt = base0 + FULL_CHUNKS * CHUNK
    pltpu.sync_copy(ei.at[0, pl.ds(bt, TAIL)], sidx_t)
    pltpu.sync_copy(ei.at[1, pl.ds(bt, TAIL)], didx_t)
    pltpu.async_copy(h.at[sidx_t], rows_t, sem).wait()
    pltpu.sync_copy(rows_t, acc_sh.at[didx_t], add=True)

    plsc.subcore_barrier()
    pltpu.sync_copy(acc_sh.at[pl.ds(s * RPT, RPT)], fl)
    pltpu.sync_copy(fl, accp.at[c, pl.ds(s * RPT, RPT)])


_prop_call = pl.kernel(
    _prop_body,
    out_type=jax.ShapeDtypeStruct((NC, N_PAD, HID), jnp.float32),
    mesh=_mesh,
    scratch_types=[
        pltpu.VMEM((CHUNK,), jnp.int32),
        pltpu.VMEM((TAIL,), jnp.int32),
        pltpu.VMEM((CHUNK,), jnp.int32),
        pltpu.VMEM((TAIL,), jnp.int32),
        pltpu.VMEM((CHUNK, HID), jnp.float32),
        pltpu.VMEM((TAIL, HID), jnp.float32),
        pltpu.VMEM((RPT, HID), jnp.float32),
        pltpu.VMEM_SHARED((N_PAD, HID), jnp.float32),
        pltpu.SemaphoreType.DMA,
    ],
)


def _lin1_body(x_ref, w_ref, degp_ref, h_ref, r_ref):
    d = degp_ref[0] + degp_ref[1] + 1.0
    r = lax.rsqrt(d)
    r_ref[...] = r
    h = jnp.dot(x_ref[...], w_ref[...], preferred_element_type=jnp.float32)
    h_ref[...] = h * r


_lin1 = pl.pallas_call(
    _lin1_body,
    grid=(N // BR,),
    in_specs=[
        pl.BlockSpec((BR, IN_CH), lambda i: (i, 0)),
        pl.BlockSpec((IN_CH, HID), lambda i: (0, 0)),
        pl.BlockSpec((NC, BR, 1), lambda i: (0, i, 0)),
    ],
    out_specs=[
        pl.BlockSpec((BR, HID), lambda i: (i, 0)),
        pl.BlockSpec((BR, 1), lambda i: (i, 0)),
    ],
    out_shape=[
        jax.ShapeDtypeStruct((N, HID), jnp.float32),
        jax.ShapeDtypeStruct((N, 1), jnp.float32),
    ],
)


def _mid_body(accp_ref, h_ref, r_ref, b_ref, w_ref, out_ref):
    r = r_ref[...]
    y = (accp_ref[0] + accp_ref[1] + h_ref[...]) * r + b_ref[...]
    u = 0.5 * y * (1.0 + lax.erf(y * 0.7071067811865476))
    out_ref[...] = jnp.dot(u, w_ref[...],
                           preferred_element_type=jnp.float32) * r


_mid = pl.pallas_call(
    _mid_body,
    grid=(N // BR,),
    in_specs=[
        pl.BlockSpec((NC, BR, HID), lambda i: (0, i, 0)),
        pl.BlockSpec((BR, HID), lambda i: (i, 0)),
        pl.BlockSpec((BR, 1), lambda i: (i, 0)),
        pl.BlockSpec((1, HID), lambda i: (0, 0)),
        pl.BlockSpec((HID, HID), lambda i: (0, 0)),
    ],
    out_specs=pl.BlockSpec((BR, HID), lambda i: (i, 0)),
    out_shape=jax.ShapeDtypeStruct((N, HID), jnp.float32),
)


def _fin_body(accp_ref, h_ref, r_ref, b_ref, out_ref):
    r = r_ref[...]
    out_ref[...] = (accp_ref[0] + accp_ref[1] + h_ref[...]) * r + b_ref[...]


_fin = pl.pallas_call(
    _fin_body,
    grid=(N // BR,),
    in_specs=[
        pl.BlockSpec((NC, BR, HID), lambda i: (0, i, 0)),
        pl.BlockSpec((BR, HID), lambda i: (i, 0)),
        pl.BlockSpec((BR, 1), lambda i: (i, 0)),
        pl.BlockSpec((1, HID), lambda i: (0, 0)),
    ],
    out_specs=pl.BlockSpec((BR, HID), lambda i: (i, 0)),
    out_shape=jax.ShapeDtypeStruct((N, HID), jnp.float32),
)


@jax.jit
def kernel(x, edge_index, W1, b1, W2, b2):
    ei = edge_index.astype(jnp.int32)
    degp = _deg_call(ei)
    degp3 = degp.reshape(NC, N_PAD, 1)
    h1, r_col = _lin1(x, W1, degp3)
    acc1 = _prop_call(h1, ei)
    h2 = _mid(acc1, h1, r_col, b1.reshape(1, HID), W2)
    acc2 = _prop_call(h2, ei)
    out = _fin(acc2, h2, r_col, b2.reshape(1, HID))
    return out


# speedup vs baseline: 20.6103x; 20.6103x over previous
import jax
import jax.numpy as jnp
from jax import lax
from jax.experimental import pallas as pl
from jax.experimental.pallas import tpu as pltpu
from jax.experimental.pallas import tpu_sc as plsc

N = 10000
E = 320000
IN_CH = 128
HID = 32
NC, NS = 2, 16
NW = NC * NS
N_PAD = 10240
RPT = N_PAD // NS
EPT = E // NW
CHUNK = 128
FULL_CHUNKS = EPT // CHUNK
TAIL = EPT - FULL_CHUNKS * CHUNK
BR = 2000

_mesh = plsc.VectorSubcoreMesh(core_axis_name="c", subcore_axis_name="s")


def _deg_body(dst_e, degp, ones_v, ones_t, didx, didx_t, zrow, deg_sh):
    c = lax.axis_index("c")
    s = lax.axis_index("s")
    wid = c * NS + s
    one16 = jnp.full((16,), 1.0, jnp.float32)
    zero16 = jnp.zeros((16,), jnp.float32)
    for j in range(CHUNK // 16):
        ones_v[pl.ds(j * 16, 16)] = one16
    ones_t[pl.ds(0, 16)] = one16

    @pl.loop(0, RPT // 16)
    def _(i):
        zrow[pl.ds(i * 16, 16)] = zero16

    pltpu.sync_copy(zrow, deg_sh.at[pl.ds(s * RPT, RPT)])
    plsc.subcore_barrier()

    base0 = wid * EPT

    @pl.loop(0, FULL_CHUNKS)
    def _(ch):
        b = base0 + ch * CHUNK
        pltpu.sync_copy(dst_e.at[pl.ds(b, CHUNK)], didx)
        pltpu.sync_copy(ones_v, deg_sh.at[didx], add=True)

    bt = base0 + FULL_CHUNKS * CHUNK
    pltpu.sync_copy(dst_e.at[pl.ds(bt, TAIL)], didx_t)
    pltpu.sync_copy(ones_t, deg_sh.at[didx_t], add=True)

    plsc.subcore_barrier()
    pltpu.sync_copy(deg_sh.at[pl.ds(s * RPT, RPT)], zrow)
    pltpu.sync_copy(zrow, degp.at[c, pl.ds(s * RPT, RPT)])


_deg_call = pl.kernel(
    _deg_body,
    out_type=jax.ShapeDtypeStruct((NC, N_PAD), jnp.float32),
    mesh=_mesh,
    scratch_types=[
        pltpu.VMEM((CHUNK,), jnp.float32),
        pltpu.VMEM((TAIL,), jnp.float32),
        pltpu.VMEM((CHUNK,), jnp.int32),
        pltpu.VMEM((TAIL,), jnp.int32),
        pltpu.VMEM((RPT,), jnp.float32),
        pltpu.VMEM_SHARED((N_PAD,), jnp.float32),
    ],
    compiler_params=pltpu.CompilerParams(use_tc_tiling_on_sc=False),
)


def _prop_body(h, src_e, dst_e, accp, sidx, sidx_t, didx, didx_t, rows, rows_t, fl,
               acc_sh, sem):
    c = lax.axis_index("c")
    s = lax.axis_index("s")
    wid = c * NS + s
    zero16 = jnp.zeros((16,), jnp.float32)

    @pl.loop(0, RPT)
    def _(i):
        fl[i, pl.ds(0, 16)] = zero16
        fl[i, pl.ds(16, 16)] = zero16

    pltpu.sync_copy(fl, acc_sh.at[pl.ds(s * RPT, RPT)])
    plsc.subcore_barrier()

    base0 = wid * EPT

    @pl.loop(0, FULL_CHUNKS)
    def _(ch):
        b = base0 + ch * CHUNK
        pltpu.sync_copy(src_e.at[pl.ds(b, CHUNK)], sidx)
        pltpu.sync_copy(dst_e.at[pl.ds(b, CHUNK)], didx)
        pltpu.async_copy(h.at[sidx], rows, sem).wait()
        pltpu.sync_copy(rows, acc_sh.at[didx], add=True)

    bt = base0 + FULL_CHUNKS * CHUNK
    pltpu.sync_copy(src_e.at[pl.ds(bt, TAIL)], sidx_t)
    pltpu.sync_copy(dst_e.at[pl.ds(bt, TAIL)], didx_t)
    pltpu.async_copy(h.at[sidx_t], rows_t, sem).wait()
    pltpu.sync_copy(rows_t, acc_sh.at[didx_t], add=True)

    plsc.subcore_barrier()
    pltpu.sync_copy(acc_sh.at[pl.ds(s * RPT, RPT)], fl)
    pltpu.sync_copy(fl, accp.at[c, pl.ds(s * RPT, RPT)])


_prop_call = pl.kernel(
    _prop_body,
    out_type=jax.ShapeDtypeStruct((NC, N_PAD, HID), jnp.float32),
    mesh=_mesh,
    scratch_types=[
        pltpu.VMEM((CHUNK,), jnp.int32),
        pltpu.VMEM((TAIL,), jnp.int32),
        pltpu.VMEM((CHUNK,), jnp.int32),
        pltpu.VMEM((TAIL,), jnp.int32),
        pltpu.VMEM((CHUNK, HID), jnp.float32),
        pltpu.VMEM((TAIL, HID), jnp.float32),
        pltpu.VMEM((RPT, HID), jnp.float32),
        pltpu.VMEM_SHARED((N_PAD, HID), jnp.float32),
        pltpu.SemaphoreType.DMA,
    ],
    compiler_params=pltpu.CompilerParams(use_tc_tiling_on_sc=False),
)


def _lin1_body(x_ref, w_ref, degp_ref, h_ref, r_ref):
    d = degp_ref[0] + degp_ref[1] + 1.0
    r = lax.rsqrt(d)
    r_ref[...] = r
    h = jnp.dot(x_ref[...], w_ref[...], preferred_element_type=jnp.float32)
    h_ref[...] = h * r


_lin1 = pl.pallas_call(
    _lin1_body,
    grid=(N // BR,),
    in_specs=[
        pl.BlockSpec((BR, IN_CH), lambda i: (i, 0)),
        pl.BlockSpec((IN_CH, HID), lambda i: (0, 0)),
        pl.BlockSpec((NC, BR, 1), lambda i: (0, i, 0)),
    ],
    out_specs=[
        pl.BlockSpec((BR, HID), lambda i: (i, 0)),
        pl.BlockSpec((BR, 1), lambda i: (i, 0)),
    ],
    out_shape=[
        jax.ShapeDtypeStruct((N, HID), jnp.float32),
        jax.ShapeDtypeStruct((N, 1), jnp.float32),
    ],
)


def _mid_body(accp_ref, h_ref, r_ref, b_ref, w_ref, out_ref):
    r = r_ref[...]
    y = (accp_ref[0] + accp_ref[1] + h_ref[...]) * r + b_ref[...]
    u = 0.5 * y * (1.0 + lax.erf(y * 0.7071067811865476))
    out_ref[...] = jnp.dot(u, w_ref[...],
                           preferred_element_type=jnp.float32) * r


_mid = pl.pallas_call(
    _mid_body,
    grid=(N // BR,),
    in_specs=[
        pl.BlockSpec((NC, BR, HID), lambda i: (0, i, 0)),
        pl.BlockSpec((BR, HID), lambda i: (i, 0)),
        pl.BlockSpec((BR, 1), lambda i: (i, 0)),
        pl.BlockSpec((1, HID), lambda i: (0, 0)),
        pl.BlockSpec((HID, HID), lambda i: (0, 0)),
    ],
    out_specs=pl.BlockSpec((BR, HID), lambda i: (i, 0)),
    out_shape=jax.ShapeDtypeStruct((N, HID), jnp.float32),
)


def _fin_body(accp_ref, h_ref, r_ref, b_ref, out_ref):
    r = r_ref[...]
    out_ref[...] = (accp_ref[0] + accp_ref[1] + h_ref[...]) * r + b_ref[...]


_fin = pl.pallas_call(
    _fin_body,
    grid=(N // BR,),
    in_specs=[
        pl.BlockSpec((NC, BR, HID), lambda i: (0, i, 0)),
        pl.BlockSpec((BR, HID), lambda i: (i, 0)),
        pl.BlockSpec((BR, 1), lambda i: (i, 0)),
        pl.BlockSpec((1, HID), lambda i: (0, 0)),
    ],
    out_specs=pl.BlockSpec((BR, HID), lambda i: (i, 0)),
    out_shape=jax.ShapeDtypeStruct((N, HID), jnp.float32),
)


@jax.jit
def kernel(x, edge_index, W1, b1, W2, b2):
    ei = edge_index.astype(jnp.int32)
    src_e = ei[0]
    dst_e = ei[1]
    degp = _deg_call(dst_e)
    degp3 = degp.reshape(NC, N_PAD, 1)
    h1, r_col = _lin1(x, W1, degp3)
    acc1 = _prop_call(h1, src_e, dst_e)
    h2 = _mid(acc1, h1, r_col, b1.reshape(1, HID), W2)
    acc2 = _prop_call(h2, src_e, dst_e)
    out = _fin(acc2, h2, r_col, b2.reshape(1, HID))
    return out


# speedup vs baseline: 24.7809x; 1.2024x over previous
import jax
import jax.numpy as jnp
from jax import lax
from jax.experimental import pallas as pl
from jax.experimental.pallas import tpu as pltpu
from jax.experimental.pallas import tpu_sc as plsc

N = 10000
E = 320000
IN_CH = 128
HID = 32
NC, NS = 2, 16
NW = NC * NS
N_PAD = 10240
RPT = N_PAD // NS
CHUNK = 128
CPT = 80
EPT = CPT * CHUNK
E_PAD = NW * EPT
BR = 2000

_mesh = plsc.VectorSubcoreMesh(core_axis_name="c", subcore_axis_name="s")


def _deg_body(dst2d, degp, ones_v, didx2d, zrow, deg_sh, sem0, sem1, sem2,
              sem3):
    c = lax.axis_index("c")
    s = lax.axis_index("s")
    wid = c * NS + s
    sems = [sem0, sem1, sem2, sem3]
    one16 = jnp.full((16,), 1.0, jnp.float32)
    zero16 = jnp.zeros((16,), jnp.float32)
    for j in range(CHUNK // 16):
        ones_v[pl.ds(j * 16, 16)] = one16

    @pl.loop(0, RPT // 16, unroll=8)
    def _(i):
        zrow[pl.ds(i * 16, 16)] = zero16

    pltpu.sync_copy(dst2d.at[pl.ds(wid * CPT, CPT)], didx2d)
    pltpu.sync_copy(zrow, deg_sh.at[pl.ds(s * RPT, RPT)])
    plsc.subcore_barrier()

    @pl.loop(0, CPT // 4)
    def _(i):
        j = i * 4
        descs = [
            pltpu.async_copy(
                ones_v, deg_sh.at[didx2d.at[j + k]], sems[k], add=True)
            for k in range(4)
        ]
        for d in descs:
            d.wait()

    plsc.subcore_barrier()
    pltpu.sync_copy(deg_sh.at[pl.ds(s * RPT, RPT)], zrow)
    pltpu.sync_copy(zrow, degp.at[c, pl.ds(s * RPT, RPT)])


_deg_call = pl.kernel(
    _deg_body,
    out_type=jax.ShapeDtypeStruct((NC, N_PAD), jnp.float32),
    mesh=_mesh,
    scratch_types=[
        pltpu.VMEM((CHUNK,), jnp.float32),
        pltpu.VMEM((CPT, CHUNK), jnp.int32),
        pltpu.VMEM((RPT,), jnp.float32),
        pltpu.VMEM_SHARED((N_PAD,), jnp.float32),
        pltpu.SemaphoreType.DMA,
        pltpu.SemaphoreType.DMA,
        pltpu.SemaphoreType.DMA,
        pltpu.SemaphoreType.DMA,
    ],
    compiler_params=pltpu.CompilerParams(use_tc_tiling_on_sc=False),
)


def _prop_body(h, src2d, dst2d, accp, sidx2d, didx2d, r0, r1, r2, r3, fl,
               acc_sh, g0, g1, g2, g3, s0, s1, s2, s3):
    c = lax.axis_index("c")
    s = lax.axis_index("s")
    wid = c * NS + s
    rows = [r0, r1, r2, r3]
    gsem = [g0, g1, g2, g3]
    ssem = [s0, s1, s2, s3]
    zero16 = jnp.zeros((16,), jnp.float32)

    @pl.loop(0, RPT, unroll=8)
    def _(i):
        fl[i, pl.ds(0, 16)] = zero16
        fl[i, pl.ds(16, 16)] = zero16

    pltpu.sync_copy(src2d.at[pl.ds(wid * CPT, CPT)], sidx2d)
    pltpu.sync_copy(dst2d.at[pl.ds(wid * CPT, CPT)], didx2d)
    pltpu.sync_copy(fl, acc_sh.at[pl.ds(s * RPT, RPT)])

    plsc.subcore_barrier()

    @pl.loop(0, CPT // 4)
    def _(i):
        j0 = i * 4
        gd = [
            pltpu.async_copy(h.at[sidx2d.at[j0 + k]], rows[k], gsem[k])
            for k in range(4)
        ]
        sd = []
        for k in range(4):
            gd[k].wait()
            sd.append(
                pltpu.async_copy(rows[k], acc_sh.at[didx2d.at[j0 + k]],
                                 ssem[k], add=True))
        for d in sd:
            d.wait()

    plsc.subcore_barrier()
    pltpu.sync_copy(acc_sh.at[pl.ds(s * RPT, RPT)], fl)
    pltpu.sync_copy(fl, accp.at[c, pl.ds(s * RPT, RPT)])


_prop_call = pl.kernel(
    _prop_body,
    out_type=jax.ShapeDtypeStruct((NC, N_PAD, HID), jnp.float32),
    mesh=_mesh,
    scratch_types=[
        pltpu.VMEM((CPT, CHUNK), jnp.int32),
        pltpu.VMEM((CPT, CHUNK), jnp.int32),
        pltpu.VMEM((CHUNK, HID), jnp.float32),
        pltpu.VMEM((CHUNK, HID), jnp.float32),
        pltpu.VMEM((CHUNK, HID), jnp.float32),
        pltpu.VMEM((CHUNK, HID), jnp.float32),
        pltpu.VMEM((RPT, HID), jnp.float32),
        pltpu.VMEM_SHARED((N_PAD, HID), jnp.float32),
        pltpu.SemaphoreType.DMA,
        pltpu.SemaphoreType.DMA,
        pltpu.SemaphoreType.DMA,
        pltpu.SemaphoreType.DMA,
        pltpu.SemaphoreType.DMA,
        pltpu.SemaphoreType.DMA,
        pltpu.SemaphoreType.DMA,
        pltpu.SemaphoreType.DMA,
    ],
    compiler_params=pltpu.CompilerParams(use_tc_tiling_on_sc=False),
)


def _lin1_body(x_ref, w_ref, degp_ref, h_ref, r_ref):
    d = degp_ref[0] + degp_ref[1] + 1.0
    r = lax.rsqrt(d)
    r_ref[...] = r
    h = jnp.dot(x_ref[...], w_ref[...], preferred_element_type=jnp.float32)
    h_ref[...] = h * r


_lin1 = pl.pallas_call(
    _lin1_body,
    grid=(N // BR,),
    in_specs=[
        pl.BlockSpec((BR, IN_CH), lambda i: (i, 0)),
        pl.BlockSpec((IN_CH, HID), lambda i: (0, 0)),
        pl.BlockSpec((NC, BR, 1), lambda i: (0, i, 0)),
    ],
    out_specs=[
        pl.BlockSpec((BR, HID), lambda i: (i, 0)),
        pl.BlockSpec((BR, 1), lambda i: (i, 0)),
    ],
    out_shape=[
        jax.ShapeDtypeStruct((N, HID), jnp.float32),
        jax.ShapeDtypeStruct((N, 1), jnp.float32),
    ],
)


def _mid_body(accp_ref, h_ref, r_ref, b_ref, w_ref, out_ref):
    r = r_ref[...]
    y = (accp_ref[0] + accp_ref[1] + h_ref[...]) * r + b_ref[...]
    u = 0.5 * y * (1.0 + lax.erf(y * 0.7071067811865476))
    out_ref[...] = jnp.dot(u, w_ref[...],
                           preferred_element_type=jnp.float32) * r


_mid = pl.pallas_call(
    _mid_body,
    grid=(N // BR,),
    in_specs=[
        pl.BlockSpec((NC, BR, HID), lambda i: (0, i, 0)),
        pl.BlockSpec((BR, HID), lambda i: (i, 0)),
        pl.BlockSpec((BR, 1), lambda i: (i, 0)),
        pl.BlockSpec((1, HID), lambda i: (0, 0)),
        pl.BlockSpec((HID, HID), lambda i: (0, 0)),
    ],
    out_specs=pl.BlockSpec((BR, HID), lambda i: (i, 0)),
    out_shape=jax.ShapeDtypeStruct((N, HID), jnp.float32),
)


def _fin_body(accp_ref, h_ref, r_ref, b_ref, out_ref):
    r = r_ref[...]
    out_ref[...] = (accp_ref[0] + accp_ref[1] + h_ref[...]) * r + b_ref[...]


_fin = pl.pallas_call(
    _fin_body,
    grid=(N // BR,),
    in_specs=[
        pl.BlockSpec((NC, BR, HID), lambda i: (0, i, 0)),
        pl.BlockSpec((BR, HID), lambda i: (i, 0)),
        pl.BlockSpec((BR, 1), lambda i: (i, 0)),
        pl.BlockSpec((1, HID), lambda i: (0, 0)),
    ],
    out_specs=pl.BlockSpec((BR, HID), lambda i: (i, 0)),
    out_shape=jax.ShapeDtypeStruct((N, HID), jnp.float32),
)


@jax.jit
def kernel(x, edge_index, W1, b1, W2, b2):
    ei = edge_index.astype(jnp.int32)
    pad = E_PAD - E
    src_p = jnp.concatenate([ei[0], jnp.zeros((pad,), jnp.int32)])
    dst_p = jnp.concatenate([ei[1], jnp.full((pad,), N, jnp.int32)])
    src2d = src_p.reshape(E_PAD // CHUNK, CHUNK)
    dst2d = dst_p.reshape(E_PAD // CHUNK, CHUNK)
    degp = _deg_call(dst2d)
    degp3 = degp.reshape(NC, N_PAD, 1)
    h1, r_col = _lin1(x, W1, degp3)
    acc1 = _prop_call(h1, src2d, dst2d)
    h2 = _mid(acc1, h1, r_col, b1.reshape(1, HID), W2)
    acc2 = _prop_call(h2, src2d, dst2d)
    out = _fin(acc2, h2, r_col, b2.reshape(1, HID))
    return out
